# baseline (speedup 1.0000x reference)
import functools

import jax
import jax.numpy as jnp
from jax import lax
from jax.experimental import pallas as pl
from jax.experimental.pallas import tpu as pltpu
from jax.experimental.pallas import tpu_sc as plsc

_L = 16

_GDN = lax.GatherDimensionNumbers(
    offset_dims=(), collapsed_slice_dims=(0,), start_index_map=(0,))


def _bcast_lane(v16, j):
    idx = jnp.full((_L, 1), j, jnp.int32)
    return lax.gather(v16, idx, dimension_numbers=_GDN, slice_sizes=(1,),
                      mode=lax.GatherScatterMode.PROMISE_IN_BOUNDS)


def _matmul(x, w):
    n, d_in = x.shape
    d_out = w.shape[1]
    blk = 1000

    def body(x_ref, w_ref, o_ref):
        o_ref[...] = jnp.dot(x_ref[...], w_ref[...],
                             preferred_element_type=jnp.float32)

    return pl.pallas_call(
        body,
        grid=(n // blk,),
        in_specs=[
            pl.BlockSpec((blk, d_in), lambda i: (i, 0)),
            pl.BlockSpec((d_in, d_out), lambda i: (0, 0)),
        ],
        out_specs=pl.BlockSpec((blk, d_out), lambda i: (i, 0)),
        out_shape=jax.ShapeDtypeStruct((n, d_out), jnp.float32),
    )(x, w)


def _merge(partials, bias):
    _, n, d = partials.shape
    blk = 1000

    def body(p_ref, b_ref, o_ref):
        o_ref[...] = p_ref[0] + p_ref[1] + b_ref[...]

    return pl.pallas_call(
        body,
        grid=(n // blk,),
        in_specs=[
            pl.BlockSpec((2, blk, d), lambda i: (0, i, 0)),
            pl.BlockSpec((1, d), lambda i: (0, 0)),
        ],
        out_specs=pl.BlockSpec((blk, d), lambda i: (i, 0)),
        out_shape=jax.ShapeDtypeStruct((n, d), jnp.float32),
    )(partials, bias)


def _spmm_partials(supp, dst_idx, src_idx, edge_weight):
    n, d = supp.shape
    e = edge_weight.shape[0]
    nw = 32
    ept = e // nw
    k = 80
    nch = ept // k
    rpt = (n // 16) // 8 * 8
    tail = n - 16 * rpt
    fpr = d // _L
    nr = 4

    mesh = plsc.VectorSubcoreMesh(core_axis_name="c", subcore_axis_name="s")

    @functools.partial(
        pl.kernel,
        out_type=jax.ShapeDtypeStruct((2, n, d), jnp.float32),
        mesh=mesh,
        scratch_types=[
            pltpu.VMEM((nr, k, d), jnp.float32),
            pltpu.VMEM((nr, k), jnp.int32),
            pltpu.VMEM((nr, k), jnp.int32),
            pltpu.VMEM((nr, k), jnp.float32),
            pltpu.VMEM_SHARED((n, d), jnp.float32),
            [pltpu.SemaphoreType.DMA] * nr,
            [pltpu.SemaphoreType.DMA] * nr,
            [pltpu.SemaphoreType.DMA] * nr,
        ],
    )
    def spmm(supp_hbm, dsti_hbm, srci_hbm, ew_hbm, part_hbm,
             rows, src_c, dst_c, w_c, acc, gsem, ssem, psem):
        c = lax.axis_index("c")
        s = lax.axis_index("s")
        wid = s * 2 + c
        base = wid * ept

        def zfill(i, carry):
            for f in range(fpr):
                rows[0, i, pl.ds(f * _L, _L)] = jnp.zeros((_L,), jnp.float32)
            return carry

        lax.fori_loop(0, k, zfill, 0)
        for r in range(rpt // k):
            pltpu.sync_copy(rows.at[0], acc.at[pl.ds(s * rpt + r * k, k)])
        rem = rpt - (rpt // k) * k
        if rem:
            pltpu.sync_copy(rows.at[0, pl.ds(0, rem)],
                            acc.at[pl.ds(s * rpt + (rpt // k) * k, rem)])

        @pl.when(s == 0)
        def _zero_tail():
            pltpu.sync_copy(rows.at[0, pl.ds(0, tail)],
                            acc.at[pl.ds(16 * rpt, tail)])

        plsc.subcore_barrier()

        def pf_idx(ci, u):
            pltpu.async_copy(srci_hbm.at[pl.ds(base + ci * k, k)],
                             src_c.at[u], psem[u])
            pltpu.async_copy(dsti_hbm.at[pl.ds(base + ci * k, k)],
                             dst_c.at[u], psem[u])
            pltpu.async_copy(ew_hbm.at[pl.ds(base + ci * k, k)],
                             w_c.at[u], psem[u])

        def pfw_idx(u):
            pltpu.make_async_copy(srci_hbm.at[pl.ds(base, k)],
                                  src_c.at[u], psem[u]).wait()
            pltpu.make_async_copy(dsti_hbm.at[pl.ds(base, k)],
                                  dst_c.at[u], psem[u]).wait()
            pltpu.make_async_copy(ew_hbm.at[pl.ds(base, k)],
                                  w_c.at[u], psem[u]).wait()

        hk = k // 2

        def start_gather(u):
            pltpu.async_copy(supp_hbm.at[src_c.at[u, pl.ds(0, hk)]],
                             rows.at[u, pl.ds(0, hk)], gsem[u])
            pltpu.async_copy(supp_hbm.at[src_c.at[u, pl.ds(hk, hk)]],
                             rows.at[u, pl.ds(hk, hk)], gsem[u])

        def wait_gather(u):
            pltpu.make_async_copy(supp_hbm.at[src_c.at[u, pl.ds(0, hk)]],
                                  rows.at[u, pl.ds(0, hk)], gsem[u]).wait()
            pltpu.make_async_copy(supp_hbm.at[src_c.at[u, pl.ds(hk, hk)]],
                                  rows.at[u, pl.ds(hk, hk)], gsem[u]).wait()

        def start_scatter(u):
            pltpu.async_copy(rows.at[u], acc.at[dst_c.at[u]], ssem[u],
                             add=True)

        def wait_scatter(u):
            pltpu.make_async_copy(rows.at[u], acc.at[dst_c.at[u]],
                                  ssem[u]).wait()

        def scale(u):
            def scale_g(g, carry):
                w16 = w_c[u, pl.ds(g * _L, _L)]
                for j in range(_L):
                    ei = g * _L + j
                    wj = _bcast_lane(w16, j)
                    for f in range(fpr):
                        sl = pl.ds(f * _L, _L)
                        rows[u, ei, sl] = rows[u, ei, sl] * wj
                return carry

            lax.fori_loop(0, k // _L, scale_g, 0)

        def body(ci, u):
            u2 = (u + 2) % nr

            @pl.when(ci >= 2)
            def _():
                wait_scatter(u2)

            @pl.when(jnp.logical_and(ci >= 2, ci + 2 < nch))
            def _():
                pf_idx(ci + 2, u2)

            u1 = (u + 1) % nr

            @pl.when(ci + 1 < nch)
            def _():
                pfw_idx(u1)
                start_gather(u1)

            wait_gather(u)
            scale(u)
            start_scatter(u)

        for u in range(nr):
            pf_idx(u, u)
        pfw_idx(0)
        start_gather(0)
        pfw_idx(1)
        start_gather(1)
        wait_gather(0)
        scale(0)
        start_scatter(0)

        def ring(i4, carry):
            ci = 1 + 4 * i4
            body(ci, 1)
            body(ci + 1, 2)
            body(ci + 2, 3)
            body(ci + 3, 0)
            return carry

        lax.fori_loop(0, (nch - 1) // 4, ring, 0)
        wait_scatter((nch - 2) % nr)
        wait_scatter((nch - 1) % nr)
        plsc.subcore_barrier()

        sl = pl.ds(s * rpt, rpt)
        pltpu.sync_copy(acc.at[sl], part_hbm.at[c, sl])

        @pl.when(s == 0)
        def _write_tail():
            tl = pl.ds(16 * rpt, tail)
            pltpu.sync_copy(acc.at[tl], part_hbm.at[c, tl])

    return spmm(supp, dst_idx, src_idx, edge_weight)


def kernel(input, edge_index, edge_weight, W, bias):
    supp = _matmul(input, W)
    partials = _spmm_partials(supp, edge_index[0], edge_index[1], edge_weight)
    return _merge(partials, bias)


# speedup vs baseline: 1.0015x; 1.0015x over previous
import functools

import jax
import jax.numpy as jnp
from jax import lax
from jax.experimental import pallas as pl
from jax.experimental.pallas import tpu as pltpu
from jax.experimental.pallas import tpu_sc as plsc

_L = 16

_GDN = lax.GatherDimensionNumbers(
    offset_dims=(), collapsed_slice_dims=(0,), start_index_map=(0,))


def _bcast_lane(v16, j):
    idx = jnp.full((_L, 1), j, jnp.int32)
    return lax.gather(v16, idx, dimension_numbers=_GDN, slice_sizes=(1,),
                      mode=lax.GatherScatterMode.PROMISE_IN_BOUNDS)


def _matmul(x, w):
    n, d_in = x.shape
    d_out = w.shape[1]
    blk = 1000

    def body(x_ref, w_ref, o_ref):
        o_ref[...] = jnp.dot(x_ref[...], w_ref[...],
                             preferred_element_type=jnp.float32)

    return pl.pallas_call(
        body,
        grid=(n // blk,),
        in_specs=[
            pl.BlockSpec((blk, d_in), lambda i: (i, 0)),
            pl.BlockSpec((d_in, d_out), lambda i: (0, 0)),
        ],
        out_specs=pl.BlockSpec((blk, d_out), lambda i: (i, 0)),
        out_shape=jax.ShapeDtypeStruct((n, d_out), jnp.float32),
    )(x, w)


def _merge(partials, bias):
    _, n, d = partials.shape
    blk = 1000

    def body(p_ref, b_ref, o_ref):
        o_ref[...] = p_ref[0] + p_ref[1] + b_ref[...]

    return pl.pallas_call(
        body,
        grid=(n // blk,),
        in_specs=[
            pl.BlockSpec((2, blk, d), lambda i: (0, i, 0)),
            pl.BlockSpec((1, d), lambda i: (0, 0)),
        ],
        out_specs=pl.BlockSpec((blk, d), lambda i: (i, 0)),
        out_shape=jax.ShapeDtypeStruct((n, d), jnp.float32),
    )(partials, bias)


def _spmm_partials(supp, dst_idx, src_idx, edge_weight):
    n, d = supp.shape
    e = edge_weight.shape[0]
    nw = 32
    ept = e // nw
    k = 80
    nch = ept // k
    rpt = (n // 16) // 8 * 8
    tail = n - 16 * rpt
    fpr = d // _L
    nr = 4

    mesh = plsc.VectorSubcoreMesh(core_axis_name="c", subcore_axis_name="s")

    @functools.partial(
        pl.kernel,
        out_type=jax.ShapeDtypeStruct((2, n, d), jnp.float32),
        mesh=mesh,
        scratch_types=[
            pltpu.VMEM((nr, k, d), jnp.float32),
            pltpu.VMEM((nr, k), jnp.int32),
            pltpu.VMEM((nr, k), jnp.int32),
            pltpu.VMEM((nr, k), jnp.float32),
            pltpu.VMEM_SHARED((n, d), jnp.float32),
            [pltpu.SemaphoreType.DMA] * nr,
            [pltpu.SemaphoreType.DMA] * nr,
            [pltpu.SemaphoreType.DMA] * nr,
        ],
    )
    def spmm(supp_hbm, dsti_hbm, srci_hbm, ew_hbm, part_hbm,
             rows, src_c, dst_c, w_c, acc, gsem, ssem, psem):
        c = lax.axis_index("c")
        s = lax.axis_index("s")
        wid = s * 2 + c
        base = wid * ept

        def zfill(i, carry):
            for f in range(fpr):
                rows[0, i, pl.ds(f * _L, _L)] = jnp.zeros((_L,), jnp.float32)
            return carry

        lax.fori_loop(0, k, zfill, 0)
        for r in range(rpt // k):
            pltpu.sync_copy(rows.at[0], acc.at[pl.ds(s * rpt + r * k, k)])
        rem = rpt - (rpt // k) * k
        if rem:
            pltpu.sync_copy(rows.at[0, pl.ds(0, rem)],
                            acc.at[pl.ds(s * rpt + (rpt // k) * k, rem)])

        @pl.when(s == 0)
        def _zero_tail():
            pltpu.sync_copy(rows.at[0, pl.ds(0, tail)],
                            acc.at[pl.ds(16 * rpt, tail)])

        plsc.subcore_barrier()

        def pf_idx(ci, u):
            pltpu.async_copy(srci_hbm.at[pl.ds(base + ci * k, k)],
                             src_c.at[u], psem[u])
            pltpu.async_copy(dsti_hbm.at[pl.ds(base + ci * k, k)],
                             dst_c.at[u], psem[u])
            pltpu.async_copy(ew_hbm.at[pl.ds(base + ci * k, k)],
                             w_c.at[u], psem[u])

        def pfw_idx(u):
            pltpu.make_async_copy(srci_hbm.at[pl.ds(base, k)],
                                  src_c.at[u], psem[u]).wait()
            pltpu.make_async_copy(dsti_hbm.at[pl.ds(base, k)],
                                  dst_c.at[u], psem[u]).wait()
            pltpu.make_async_copy(ew_hbm.at[pl.ds(base, k)],
                                  w_c.at[u], psem[u]).wait()

        def start_gather(u):
            pltpu.async_copy(supp_hbm.at[src_c.at[u]], rows.at[u], gsem[u])

        def wait_gather(u):
            pltpu.make_async_copy(supp_hbm.at[src_c.at[u]], rows.at[u],
                                  gsem[u]).wait()

        def start_scatter(u):
            pltpu.async_copy(rows.at[u], acc.at[dst_c.at[u]], ssem[u],
                             add=True)

        def wait_scatter(u):
            pltpu.make_async_copy(rows.at[u], acc.at[dst_c.at[u]],
                                  ssem[u]).wait()

        def scale(u):
            def scale_g(g, carry):
                w16 = w_c[u, pl.ds(g * _L, _L)]
                for j in range(_L):
                    ei = g * _L + j
                    wj = _bcast_lane(w16, j)
                    for f in range(fpr):
                        sl = pl.ds(f * _L, _L)
                        rows[u, ei, sl] = rows[u, ei, sl] * wj
                return carry

            lax.fori_loop(0, k // _L, scale_g, 0)

        def body(ci, u):
            u2 = (u + 2) % nr

            @pl.when(ci >= 2)
            def _():
                wait_scatter(u2)

            @pl.when(jnp.logical_and(ci >= 2, ci + 2 < nch))
            def _():
                pf_idx(ci + 2, u2)

            u1 = (u + 1) % nr

            @pl.when(ci + 1 < nch)
            def _():
                pfw_idx(u1)
                start_gather(u1)

            wait_gather(u)
            scale(u)
            start_scatter(u)

        for u in range(nr):
            pf_idx(u, u)
        pfw_idx(0)
        start_gather(0)
        pfw_idx(1)
        start_gather(1)
        wait_gather(0)
        scale(0)
        start_scatter(0)

        def ring(i4, carry):
            ci = 1 + 4 * i4
            body(ci, 1)
            body(ci + 1, 2)
            body(ci + 2, 3)
            body(ci + 3, 0)
            return carry

        lax.fori_loop(0, (nch - 1) // 4, ring, 0)
        wait_scatter((nch - 2) % nr)
        wait_scatter((nch - 1) % nr)
        plsc.subcore_barrier()

        sl = pl.ds(s * rpt, rpt)
        pltpu.sync_copy(acc.at[sl], part_hbm.at[c, sl])

        @pl.when(s == 0)
        def _write_tail():
            tl = pl.ds(16 * rpt, tail)
            pltpu.sync_copy(acc.at[tl], part_hbm.at[c, tl])

    return spmm(supp, dst_idx, src_idx, edge_weight)


def kernel(input, edge_index, edge_weight, W, bias):
    supp = _matmul(input, W)
    partials = _spmm_partials(supp, edge_index[0], edge_index[1], edge_weight)
    return _merge(partials, bias)
